# baseline (speedup 1.0000x reference)
import jax
import jax.numpy as jnp
from jax import lax
from jax.experimental import pallas as pl

_SEQ = 128
_D = 64
_C = 64
_BS = 256
_BQ = 128


def _pool_project(x, w):
    pooled = jnp.sum(x, axis=2) * (1.0 / _SEQ)
    return jnp.dot(pooled, w, preferred_element_type=jnp.float32)


def _body(nbs, labels_ref, xs_ref, xq_ref, w_ref, b_ref,
          sums_ref, counts_ref, logits_t_ref, protos_ref):
    i = pl.program_id(0)

    @pl.when(i < nbs)
    def _support_phase():
        emb = _pool_project(xs_ref[...], w_ref[...])
        lbl = labels_ref[0, 0, :]
        onehot = (lbl[:, None] ==
                  lax.broadcasted_iota(jnp.int32, (_BS, _C), 1)
                  ).astype(jnp.float32)
        part_sums = lax.dot_general(onehot, emb, (((0,), (0,)), ((), ())),
                                    preferred_element_type=jnp.float32)
        ones_col = jnp.ones((_BS, 1), jnp.float32)
        part_counts = lax.dot_general(onehot, ones_col,
                                      (((0,), (0,)), ((), ())),
                                      preferred_element_type=jnp.float32)

        @pl.when(i == 0)
        def _():
            sums_ref[...] = part_sums
            counts_ref[...] = part_counts

        @pl.when(i > 0)
        def _():
            sums_ref[...] += part_sums
            counts_ref[...] += part_counts

    @pl.when(i >= nbs)
    def _query_phase():
        counts = counts_ref[...]
        denom = jnp.maximum(counts, 1.0)
        protos = (sums_ref[...] + counts * b_ref[...]) / denom

        @pl.when(i == nbs)
        def _():
            protos_ref[...] = protos

        qe = _pool_project(xq_ref[...], w_ref[...]) + b_ref[...]
        p2 = jnp.sum(protos * protos, axis=1, keepdims=True)
        ones_row = jnp.ones((1, _D), jnp.float32)
        q2t = lax.dot_general(ones_row, qe * qe, (((1,), (1,)), ((), ())),
                              preferred_element_type=jnp.float32)
        cross_t = lax.dot_general(protos, qe, (((1,), (1,)), ((), ())),
                                  preferred_element_type=jnp.float32)
        logits_t_ref[...] = -(p2 + q2t - 2.0 * cross_t + 1e-8)


@jax.jit
def kernel(support, support_labels, query, W, b):
    n_sup = support.shape[0]
    n_q = query.shape[0]
    nbs = n_sup // _BS
    nbq = n_q // _BQ
    sup_t = support.transpose(0, 2, 1)
    q_t = query.transpose(0, 2, 1)
    labels = support_labels.astype(jnp.int32).reshape(nbs, 1, _BS)
    b_row = b.reshape(1, _D)

    import functools
    body = functools.partial(_body, nbs)

    logits_t, protos = pl.pallas_call(
        body,
        grid=(nbs + nbq,),
        in_specs=[
            pl.BlockSpec((1, 1, _BS), lambda i: (jnp.minimum(i, nbs - 1), 0, 0)),
            pl.BlockSpec((_BS, _D, _SEQ),
                         lambda i: (jnp.minimum(i, nbs - 1), 0, 0)),
            pl.BlockSpec((_BQ, _D, _SEQ),
                         lambda i: (jnp.maximum(i - nbs, 0), 0, 0)),
            pl.BlockSpec((_D, _D), lambda i: (0, 0)),
            pl.BlockSpec((1, _D), lambda i: (0, 0)),
        ],
        out_specs=[
            pl.BlockSpec((_C, _D), lambda i: (0, 0)),
            pl.BlockSpec((_C, 1), lambda i: (0, 0)),
            pl.BlockSpec((_C, _BQ), lambda i: (0, jnp.maximum(i - nbs, 0))),
            pl.BlockSpec((_C, _D), lambda i: (0, 0)),
        ],
        out_shape=[
            jax.ShapeDtypeStruct((_C, _D), jnp.float32),
            jax.ShapeDtypeStruct((_C, 1), jnp.float32),
            jax.ShapeDtypeStruct((_C, n_q), jnp.float32),
            jax.ShapeDtypeStruct((_C, _D), jnp.float32),
        ],
    )(labels, sup_t, q_t, W, b_row)[2:]

    return (logits_t.T, protos)
